# Initial kernel scaffold; baseline (speedup 1.0000x reference)
import jax
import jax.numpy as jnp
from jax.experimental import pallas as pl


def kernel(x, edge_index, W, b, bn_gamma, bn_beta, lin_W, lin_b):
    raise NotImplementedError("write your pallas kernel here")


# speedup vs baseline: 6.3334x; 6.3334x over previous
import numpy as np
import jax
import jax.numpy as jnp
from jax import lax
from jax.experimental import pallas as pl
from jax.experimental.pallas import tpu as pltpu
from jax.experimental.pallas import tpu_sc as plsc

_N = 10000
_E = 320000
_D = 128
_OUT = 64
_L = 5
_ALPHA = 0.1
_BETA = float(np.log(1.0 / 128.0 + 1.0))
_EPS = 1e-5

_NPAD = 10240
_NC, _NS = 2, 16
_NW = _NC * _NS
_EPT = _E // _NW
_K = 80
_NCH = _EPT // _K
_RPT = _NPAD // _NS
_NZB = _RPT // _K

def _agg_body(feat, src3, dst3, zrows, out, idx_s, idx_d, rows, acc, sem):
    c = lax.axis_index("c")
    s = lax.axis_index("s")
    wid = s * _NC + c
    rbase = s * _RPT
    pltpu.sync_copy(zrows, rows)
    for i in range(_NZB):
        pltpu.sync_copy(rows, acc.at[pl.ds(rbase + i * _K, _K)])
    pltpu.sync_copy(src3.at[wid], idx_s)
    pltpu.sync_copy(dst3.at[wid], idx_d)
    plsc.subcore_barrier()

    def chunk(j, carry):
        pltpu.async_copy(feat.at[idx_s.at[j]], rows, sem).wait()
        pltpu.sync_copy(rows, acc.at[idx_d.at[j]], add=True)
        return carry

    lax.fori_loop(0, _NCH, chunk, 0)
    plsc.subcore_barrier()
    for i in range(_NZB):
        pltpu.sync_copy(acc.at[pl.ds(rbase + i * _K, _K)], rows)
        pltpu.sync_copy(rows, out.at[c, pl.ds(rbase + i * _K, _K)])


import functools


@functools.cache
def _sc_mesh():
    return plsc.VectorSubcoreMesh(
        core_axis_name="c", subcore_axis_name="s",
        num_cores=_NC, num_subcores=_NS)


@functools.cache
def _agg_kernel():
    return pl.kernel(
        _agg_body,
        out_type=jax.ShapeDtypeStruct((_NC, _NPAD, _D), jnp.float32),
        mesh=_sc_mesh(),
        scratch_types=[
            pltpu.VMEM((_NCH, _K), jnp.int32),
            pltpu.VMEM((_NCH, _K), jnp.int32),
            pltpu.VMEM((_K, _D), jnp.float32),
            pltpu.VMEM_SHARED((_NPAD, _D), jnp.float32),
            pltpu.SemaphoreType.DMA,
        ],
    )


def _agg(feat, src3, dst3, zrows):
    return _agg_kernel()(feat, src3, dst3, zrows)


def _deg_body(dst3, ones_h, zer_h, out, idx_d, ones_v, buf, acc):
    c = lax.axis_index("c")
    s = lax.axis_index("s")
    wid = s * _NC + c
    rbase = s * _RPT
    pltpu.sync_copy(zer_h, buf)
    for i in range(_NZB):
        pltpu.sync_copy(buf, acc.at[pl.ds(rbase + i * _K, _K)])
    pltpu.sync_copy(ones_h, ones_v)
    pltpu.sync_copy(dst3.at[wid], idx_d)
    plsc.subcore_barrier()

    def chunk(j, carry):
        pltpu.sync_copy(ones_v, acc.at[idx_d.at[j]], add=True)
        return carry

    lax.fori_loop(0, _NCH, chunk, 0)
    plsc.subcore_barrier()
    for i in range(_NZB):
        pltpu.sync_copy(acc.at[pl.ds(rbase + i * _K, _K)], buf)
        pltpu.sync_copy(buf, out.at[c, pl.ds(rbase + i * _K, _K)])


@functools.cache
def _deg_kernel():
    return pl.kernel(
        _deg_body,
        out_type=jax.ShapeDtypeStruct((_NC, _NPAD, _D), jnp.float32),
        mesh=_sc_mesh(),
        scratch_types=[
            pltpu.VMEM((_NCH, _K), jnp.int32),
            pltpu.VMEM((_K, _D), jnp.float32),
            pltpu.VMEM((_K, _D), jnp.float32),
            pltpu.VMEM_SHARED((_NPAD, _D), jnp.float32),
        ],
    )


def _deg(dst3, ones_h, zer_h):
    return _deg_kernel()(dst3, ones_h, zer_h)


_R = 1000
_NB = _N // _R

_tc_params = pltpu.CompilerParams(dimension_semantics=("arbitrary",))


def _c0_body(x_ref, d0_ref, d1_ref, norm_ref, feat_ref, pool_ref):
    i = pl.program_id(0)
    deg = d0_ref[:, 0:1] + d1_ref[:, 0:1]
    nrm = lax.rsqrt(jnp.maximum(deg, 1.0))
    norm_ref[...] = nrm
    xv = x_ref[...]
    feat_ref[...] = xv * nrm

    @pl.when(i == 0)
    def _():
        pool_ref[...] = jnp.zeros_like(pool_ref)

    pool_ref[...] += jnp.sum(xv, axis=0, keepdims=True)


def _run_c0(x, deg0, deg1):
    return pl.pallas_call(
        _c0_body,
        grid=(_NB,),
        in_specs=[
            pl.BlockSpec((_R, _D), lambda i: (i, 0)),
            pl.BlockSpec((_R, _D), lambda i: (i, 0)),
            pl.BlockSpec((_R, _D), lambda i: (i, 0)),
        ],
        out_specs=[
            pl.BlockSpec((_R, 1), lambda i: (i, 0)),
            pl.BlockSpec((_R, _D), lambda i: (i, 0)),
            pl.BlockSpec((1, _D), lambda i: (0, 0)),
        ],
        out_shape=[
            jax.ShapeDtypeStruct((_NPAD, 1), jnp.float32),
            jax.ShapeDtypeStruct((_NPAD, _D), jnp.float32),
            jax.ShapeDtypeStruct((1, _D), jnp.float32),
        ],
        compiler_params=_tc_params,
    )(x, deg0, deg1)


def _c1_body(h_ref, a0_ref, a1_ref, nrm_ref, w_ref, b_ref,
             rst_ref, ssum_ref, ssq_ref):
    i = pl.program_id(0)
    ag = (a0_ref[...] + a1_ref[...]) * nrm_ref[...]
    sup = (1.0 - _ALPHA) * ag + _ALPHA * h_ref[...]
    rst = ((1.0 - _BETA) * sup
           + _BETA * jnp.dot(sup, w_ref[...],
                             preferred_element_type=jnp.float32)
           + b_ref[...])
    rst_ref[...] = rst

    @pl.when(i == 0)
    def _():
        ssum_ref[...] = jnp.zeros_like(ssum_ref)
        ssq_ref[...] = jnp.zeros_like(ssq_ref)

    ssum_ref[...] += jnp.sum(rst, axis=0, keepdims=True)
    ssq_ref[...] += jnp.sum(rst * rst, axis=0, keepdims=True)


def _run_c1(h, a0, a1, norm, w, b2):
    return pl.pallas_call(
        _c1_body,
        grid=(_NB,),
        in_specs=[
            pl.BlockSpec((_R, _D), lambda i: (i, 0)),
            pl.BlockSpec((_R, _D), lambda i: (i, 0)),
            pl.BlockSpec((_R, _D), lambda i: (i, 0)),
            pl.BlockSpec((_R, 1), lambda i: (i, 0)),
            pl.BlockSpec((_D, _D), lambda i: (0, 0)),
            pl.BlockSpec((1, _D), lambda i: (0, 0)),
        ],
        out_specs=[
            pl.BlockSpec((_R, _D), lambda i: (i, 0)),
            pl.BlockSpec((1, _D), lambda i: (0, 0)),
            pl.BlockSpec((1, _D), lambda i: (0, 0)),
        ],
        out_shape=[
            jax.ShapeDtypeStruct((_N, _D), jnp.float32),
            jax.ShapeDtypeStruct((1, _D), jnp.float32),
            jax.ShapeDtypeStruct((1, _D), jnp.float32),
        ],
        compiler_params=_tc_params,
    )(h, a0, a1, norm, w, b2)


def _c2_body(rst_ref, ssum_ref, ssq_ref, g_ref, be_ref, nrm_ref,
             h_ref, feat_ref, pool_ref):
    i = pl.program_id(0)
    mean = ssum_ref[...] * (1.0 / _N)
    var = ssq_ref[...] * (1.0 / _N) - mean * mean
    inv = lax.rsqrt(var + _EPS)
    hn = (rst_ref[...] - mean) * inv * g_ref[...] + be_ref[...]
    h = jnp.maximum(hn, 0.0)
    h_ref[...] = h
    feat_ref[...] = h * nrm_ref[...]

    @pl.when(i == 0)
    def _():
        pool_ref[...] = jnp.zeros_like(pool_ref)

    pool_ref[...] += jnp.sum(h, axis=0, keepdims=True)


def _run_c2(rst, ssum, ssq, g2, be2, norm):
    return pl.pallas_call(
        _c2_body,
        grid=(_NB,),
        in_specs=[
            pl.BlockSpec((_R, _D), lambda i: (i, 0)),
            pl.BlockSpec((1, _D), lambda i: (0, 0)),
            pl.BlockSpec((1, _D), lambda i: (0, 0)),
            pl.BlockSpec((1, _D), lambda i: (0, 0)),
            pl.BlockSpec((1, _D), lambda i: (0, 0)),
            pl.BlockSpec((_R, 1), lambda i: (i, 0)),
        ],
        out_specs=[
            pl.BlockSpec((_R, _D), lambda i: (i, 0)),
            pl.BlockSpec((_R, _D), lambda i: (i, 0)),
            pl.BlockSpec((1, _D), lambda i: (0, 0)),
        ],
        out_shape=[
            jax.ShapeDtypeStruct((_N, _D), jnp.float32),
            jax.ShapeDtypeStruct((_NPAD, _D), jnp.float32),
            jax.ShapeDtypeStruct((1, _D), jnp.float32),
        ],
        compiler_params=_tc_params,
    )(rst, ssum, ssq, g2, be2, norm)


def _d_body(pf_ref, p5_ref, lw_ref, lb_ref, out1_ref, out2_ref):
    s = (jnp.dot(pf_ref[...], lw_ref[...],
                 preferred_element_type=jnp.float32)
         + jnp.sum(lb_ref[...], axis=0, keepdims=True))
    m = jnp.max(s, axis=-1, keepdims=True)
    e = jnp.exp(s - m)
    lse = jnp.log(jnp.sum(e, axis=-1, keepdims=True))
    out1_ref[...] = s - m - lse
    out2_ref[...] = jnp.mean(p5_ref[...], axis=0, keepdims=True)


def _run_d(pf, p5, lwt, lb):
    return pl.pallas_call(
        _d_body,
        out_shape=[
            jax.ShapeDtypeStruct((1, _OUT), jnp.float32),
            jax.ShapeDtypeStruct((1, _D), jnp.float32),
        ],
    )(pf, p5, lwt, lb)


def kernel(x, edge_index, W, b, bn_gamma, bn_beta, lin_W, lin_b):
    src3 = edge_index[0].reshape(_NW, _NCH, _K)
    dst3 = edge_index[1].reshape(_NW, _NCH, _K)
    zrows = jnp.zeros((_K, _D), jnp.float32)
    orows = jnp.ones((_K, _D), jnp.float32)

    degp = _deg(dst3, orows, zrows)
    norm, feat, pool0 = _run_c0(x, degp[0, :_N], degp[1, :_N])

    h = x
    pooled = [pool0]
    for l in range(_L):
        aggp = _agg(feat, src3, dst3, zrows)
        rst, ssum, ssq = _run_c1(h, aggp[0, :_N], aggp[1, :_N],
                                 norm[:_N], W[l], b[l][None])
        h, feat, pool = _run_c2(rst, ssum, ssq, bn_gamma[l][None],
                                bn_beta[l][None], norm[:_N])
        pooled.append(pool)

    pf = jnp.concatenate(pooled, axis=1)
    p5 = jnp.concatenate(pooled[1:], axis=0)
    lwt = jnp.transpose(lin_W, (0, 2, 1)).reshape((_L + 1) * _D, _OUT)
    out1, out2 = _run_d(pf, p5, lwt, lin_b)
    return out1, out2


# speedup vs baseline: 7.8073x; 1.2327x over previous
import numpy as np
import jax
import jax.numpy as jnp
from jax import lax
from jax.experimental import pallas as pl
from jax.experimental.pallas import tpu as pltpu
from jax.experimental.pallas import tpu_sc as plsc

_N = 10000
_E = 320000
_D = 128
_OUT = 64
_L = 5
_ALPHA = 0.1
_BETA = float(np.log(1.0 / 128.0 + 1.0))
_EPS = 1e-5

_NPAD = 10240
_NC, _NS = 2, 16
_NW = _NC * _NS
_EPT = _E // _NW
_K = 80
_NCH = _EPT // _K
_RPT = _NPAD // _NS
_NZB = _RPT // _K

def _agg_body(feat, src, dst3, zrows, out,
              idx_d, ia, ib, rows_a, rows_b, acc,
              sem_ia, sem_ib, sem_ga, sem_gb, sem_sa, sem_sb):
    assert _NCH % 2 == 1
    c = lax.axis_index("c")
    s = lax.axis_index("s")
    wid = s * _NC + c
    rbase = s * _RPT
    ebase = wid * _EPT
    pltpu.sync_copy(zrows, rows_a)
    for i in range(_NZB):
        pltpu.sync_copy(rows_a, acc.at[pl.ds(rbase + i * _K, _K)])
    pltpu.sync_copy(dst3.at[wid], idx_d)
    plsc.subcore_barrier()

    def il(j, buf, sem):
        pltpu.async_copy(src.at[pl.ds(ebase + j * _K, _K)], buf, sem)

    def il_wait(buf, sem):
        pltpu.make_async_copy(src.at[pl.ds(0, _K)], buf, sem).wait()

    def g(buf_i, buf, sem):
        pltpu.async_copy(feat.at[buf_i], buf, sem)

    def g_wait(buf, sem):
        pltpu.make_async_copy(feat.at[ia], buf, sem).wait()

    def sct(j, buf, sem):
        pltpu.async_copy(buf, acc.at[idx_d.at[j]], sem, add=True)

    def s_wait(buf, sem):
        pltpu.make_async_copy(buf, acc.at[idx_d.at[0]], sem).wait()

    il(0, ia, sem_ia)
    il_wait(ia, sem_ia)
    g(ia, rows_a, sem_ga)
    il(1, ib, sem_ib)

    def pair(t, carry):
        j0 = 2 * t
        j1 = j0 + 1
        j2 = j0 + 2
        j3 = j0 + 3
        g_wait(rows_a, sem_ga)

        @pl.when(j1 < _NCH)
        def _():
            il_wait(ib, sem_ib)

        @pl.when(t > 0)
        def _():
            s_wait(rows_b, sem_sb)

        sct(j0, rows_a, sem_sa)

        @pl.when(j1 < _NCH)
        def _():
            g(ib, rows_b, sem_gb)

        @pl.when(j2 < _NCH)
        def _():
            il(j2, ia, sem_ia)

        @pl.when(j1 < _NCH)
        def _():
            g_wait(rows_b, sem_gb)

        s_wait(rows_a, sem_sa)

        @pl.when(j1 < _NCH)
        def _():
            sct(j1, rows_b, sem_sb)

        @pl.when(j2 < _NCH)
        def _():
            il_wait(ia, sem_ia)
            g(ia, rows_a, sem_ga)

        @pl.when(j3 < _NCH)
        def _():
            il(j3, ib, sem_ib)

        return carry

    lax.fori_loop(0, (_NCH + 1) // 2, pair, 0)
    plsc.subcore_barrier()
    for i in range(_NZB):
        pltpu.sync_copy(acc.at[pl.ds(rbase + i * _K, _K)], rows_a)
        pltpu.sync_copy(rows_a, out.at[c, pl.ds(rbase + i * _K, _K)])


import functools


@functools.cache
def _sc_mesh():
    return plsc.VectorSubcoreMesh(
        core_axis_name="c", subcore_axis_name="s",
        num_cores=_NC, num_subcores=_NS)


@functools.cache
def _agg_kernel():
    return pl.kernel(
        _agg_body,
        out_type=jax.ShapeDtypeStruct((_NC, _NPAD, _D), jnp.float32),
        mesh=_sc_mesh(),
        scratch_types=[
            pltpu.VMEM((_NCH, _K), jnp.int32),
            pltpu.VMEM((_K,), jnp.int32),
            pltpu.VMEM((_K,), jnp.int32),
            pltpu.VMEM((_K, _D), jnp.float32),
            pltpu.VMEM((_K, _D), jnp.float32),
            pltpu.VMEM_SHARED((_NPAD, _D), jnp.float32),
            pltpu.SemaphoreType.DMA,
            pltpu.SemaphoreType.DMA,
            pltpu.SemaphoreType.DMA,
            pltpu.SemaphoreType.DMA,
            pltpu.SemaphoreType.DMA,
            pltpu.SemaphoreType.DMA,
        ],
    )


def _agg(feat, src, dst3, zrows):
    return _agg_kernel()(feat, src, dst3, zrows)


def _deg_body(dst3, ones_h, zer_h, out, idx_d, ones_v, buf, acc,
              sem_sa, sem_sb):
    c = lax.axis_index("c")
    s = lax.axis_index("s")
    wid = s * _NC + c
    rbase = s * _RPT
    pltpu.sync_copy(zer_h, buf)
    for i in range(_NZB):
        pltpu.sync_copy(buf, acc.at[pl.ds(rbase + i * _K, _K)])
    pltpu.sync_copy(ones_h, ones_v)
    pltpu.sync_copy(dst3.at[wid], idx_d)
    plsc.subcore_barrier()

    def pair(t, carry):
        j0 = 2 * t
        j1 = j0 + 1

        @pl.when(t > 0)
        def _():
            pltpu.make_async_copy(ones_v, acc.at[idx_d.at[0]], sem_sa).wait()

        pltpu.async_copy(ones_v, acc.at[idx_d.at[j0]], sem_sa, add=True)

        @pl.when(t > 0)
        def _():
            pltpu.make_async_copy(ones_v, acc.at[idx_d.at[0]], sem_sb).wait()

        @pl.when(j1 < _NCH)
        def _():
            pltpu.async_copy(ones_v, acc.at[idx_d.at[j1]], sem_sb, add=True)

        return carry

    lax.fori_loop(0, (_NCH + 1) // 2, pair, 0)
    pltpu.make_async_copy(ones_v, acc.at[idx_d.at[0]], sem_sa).wait()
    if _NCH % 2 == 0:
        pltpu.make_async_copy(ones_v, acc.at[idx_d.at[0]], sem_sb).wait()
    plsc.subcore_barrier()
    for i in range(_NZB):
        pltpu.sync_copy(acc.at[pl.ds(rbase + i * _K, _K)], buf)
        pltpu.sync_copy(buf, out.at[c, pl.ds(rbase + i * _K, _K)])


@functools.cache
def _deg_kernel():
    return pl.kernel(
        _deg_body,
        out_type=jax.ShapeDtypeStruct((_NC, _NPAD, _D), jnp.float32),
        mesh=_sc_mesh(),
        scratch_types=[
            pltpu.VMEM((_NCH, _K), jnp.int32),
            pltpu.VMEM((_K, _D), jnp.float32),
            pltpu.VMEM((_K, _D), jnp.float32),
            pltpu.VMEM_SHARED((_NPAD, _D), jnp.float32),
            pltpu.SemaphoreType.DMA,
            pltpu.SemaphoreType.DMA,
        ],
    )


def _deg(dst3, ones_h, zer_h):
    return _deg_kernel()(dst3, ones_h, zer_h)


_R = 1000
_NB = _N // _R

_tc_params = pltpu.CompilerParams(dimension_semantics=("arbitrary",))


def _c0_body(x_ref, d0_ref, d1_ref, norm_ref, feat_ref, pool_ref):
    i = pl.program_id(0)
    deg = d0_ref[:, 0:1] + d1_ref[:, 0:1]
    nrm = lax.rsqrt(jnp.maximum(deg, 1.0))
    norm_ref[...] = nrm
    xv = x_ref[...]
    feat_ref[...] = xv * nrm

    @pl.when(i == 0)
    def _():
        pool_ref[...] = jnp.zeros_like(pool_ref)

    pool_ref[...] += jnp.sum(xv, axis=0, keepdims=True)


def _run_c0(x, deg0, deg1):
    return pl.pallas_call(
        _c0_body,
        grid=(_NB,),
        in_specs=[
            pl.BlockSpec((_R, _D), lambda i: (i, 0)),
            pl.BlockSpec((_R, _D), lambda i: (i, 0)),
            pl.BlockSpec((_R, _D), lambda i: (i, 0)),
        ],
        out_specs=[
            pl.BlockSpec((_R, 1), lambda i: (i, 0)),
            pl.BlockSpec((_R, _D), lambda i: (i, 0)),
            pl.BlockSpec((1, _D), lambda i: (0, 0)),
        ],
        out_shape=[
            jax.ShapeDtypeStruct((_NPAD, 1), jnp.float32),
            jax.ShapeDtypeStruct((_NPAD, _D), jnp.float32),
            jax.ShapeDtypeStruct((1, _D), jnp.float32),
        ],
        compiler_params=_tc_params,
    )(x, deg0, deg1)


def _c1_body(h_ref, a0_ref, a1_ref, nrm_ref, w_ref, b_ref,
             rst_ref, ssum_ref, ssq_ref):
    i = pl.program_id(0)
    ag = (a0_ref[...] + a1_ref[...]) * nrm_ref[...]
    sup = (1.0 - _ALPHA) * ag + _ALPHA * h_ref[...]
    rst = ((1.0 - _BETA) * sup
           + _BETA * jnp.dot(sup, w_ref[...],
                             preferred_element_type=jnp.float32)
           + b_ref[...])
    rst_ref[...] = rst

    @pl.when(i == 0)
    def _():
        ssum_ref[...] = jnp.zeros_like(ssum_ref)
        ssq_ref[...] = jnp.zeros_like(ssq_ref)

    ssum_ref[...] += jnp.sum(rst, axis=0, keepdims=True)
    ssq_ref[...] += jnp.sum(rst * rst, axis=0, keepdims=True)


def _run_c1(h, a0, a1, norm, w, b2):
    return pl.pallas_call(
        _c1_body,
        grid=(_NB,),
        in_specs=[
            pl.BlockSpec((_R, _D), lambda i: (i, 0)),
            pl.BlockSpec((_R, _D), lambda i: (i, 0)),
            pl.BlockSpec((_R, _D), lambda i: (i, 0)),
            pl.BlockSpec((_R, 1), lambda i: (i, 0)),
            pl.BlockSpec((_D, _D), lambda i: (0, 0)),
            pl.BlockSpec((1, _D), lambda i: (0, 0)),
        ],
        out_specs=[
            pl.BlockSpec((_R, _D), lambda i: (i, 0)),
            pl.BlockSpec((1, _D), lambda i: (0, 0)),
            pl.BlockSpec((1, _D), lambda i: (0, 0)),
        ],
        out_shape=[
            jax.ShapeDtypeStruct((_N, _D), jnp.float32),
            jax.ShapeDtypeStruct((1, _D), jnp.float32),
            jax.ShapeDtypeStruct((1, _D), jnp.float32),
        ],
        compiler_params=_tc_params,
    )(h, a0, a1, norm, w, b2)


def _c2_body(rst_ref, ssum_ref, ssq_ref, g_ref, be_ref, nrm_ref,
             h_ref, feat_ref, pool_ref):
    i = pl.program_id(0)
    mean = ssum_ref[...] * (1.0 / _N)
    var = ssq_ref[...] * (1.0 / _N) - mean * mean
    inv = lax.rsqrt(var + _EPS)
    hn = (rst_ref[...] - mean) * inv * g_ref[...] + be_ref[...]
    h = jnp.maximum(hn, 0.0)
    h_ref[...] = h
    feat_ref[...] = h * nrm_ref[...]

    @pl.when(i == 0)
    def _():
        pool_ref[...] = jnp.zeros_like(pool_ref)

    pool_ref[...] += jnp.sum(h, axis=0, keepdims=True)


def _run_c2(rst, ssum, ssq, g2, be2, norm):
    return pl.pallas_call(
        _c2_body,
        grid=(_NB,),
        in_specs=[
            pl.BlockSpec((_R, _D), lambda i: (i, 0)),
            pl.BlockSpec((1, _D), lambda i: (0, 0)),
            pl.BlockSpec((1, _D), lambda i: (0, 0)),
            pl.BlockSpec((1, _D), lambda i: (0, 0)),
            pl.BlockSpec((1, _D), lambda i: (0, 0)),
            pl.BlockSpec((_R, 1), lambda i: (i, 0)),
        ],
        out_specs=[
            pl.BlockSpec((_R, _D), lambda i: (i, 0)),
            pl.BlockSpec((_R, _D), lambda i: (i, 0)),
            pl.BlockSpec((1, _D), lambda i: (0, 0)),
        ],
        out_shape=[
            jax.ShapeDtypeStruct((_N, _D), jnp.float32),
            jax.ShapeDtypeStruct((_NPAD, _D), jnp.float32),
            jax.ShapeDtypeStruct((1, _D), jnp.float32),
        ],
        compiler_params=_tc_params,
    )(rst, ssum, ssq, g2, be2, norm)


def _d_body(pf_ref, p5_ref, lw_ref, lb_ref, out1_ref, out2_ref):
    s = (jnp.dot(pf_ref[...], lw_ref[...],
                 preferred_element_type=jnp.float32)
         + jnp.sum(lb_ref[...], axis=0, keepdims=True))
    m = jnp.max(s, axis=-1, keepdims=True)
    e = jnp.exp(s - m)
    lse = jnp.log(jnp.sum(e, axis=-1, keepdims=True))
    out1_ref[...] = s - m - lse
    out2_ref[...] = jnp.mean(p5_ref[...], axis=0, keepdims=True)


def _run_d(pf, p5, lwt, lb):
    return pl.pallas_call(
        _d_body,
        out_shape=[
            jax.ShapeDtypeStruct((1, _OUT), jnp.float32),
            jax.ShapeDtypeStruct((1, _D), jnp.float32),
        ],
    )(pf, p5, lwt, lb)


def kernel(x, edge_index, W, b, bn_gamma, bn_beta, lin_W, lin_b):
    src = edge_index[0]
    dst3 = edge_index[1].reshape(_NW, _NCH, _K)
    zrows = jnp.zeros((_K, _D), jnp.float32)
    orows = jnp.ones((_K, _D), jnp.float32)

    degp = _deg(dst3, orows, zrows)
    norm, feat, pool0 = _run_c0(x, degp[0, :_N], degp[1, :_N])

    h = x
    pooled = [pool0]
    for l in range(_L):
        aggp = _agg(feat, src, dst3, zrows)
        rst, ssum, ssq = _run_c1(h, aggp[0, :_N], aggp[1, :_N],
                                 norm[:_N], W[l], b[l][None])
        h, feat, pool = _run_c2(rst, ssum, ssq, bn_gamma[l][None],
                                bn_beta[l][None], norm[:_N])
        pooled.append(pool)

    pf = jnp.concatenate(pooled, axis=1)
    p5 = jnp.concatenate(pooled[1:], axis=0)
    lwt = jnp.transpose(lin_W, (0, 2, 1)).reshape((_L + 1) * _D, _OUT)
    out1, out2 = _run_d(pf, p5, lwt, lin_b)
    return out1, out2


# speedup vs baseline: 7.9505x; 1.0184x over previous
import numpy as np
import jax
import jax.numpy as jnp
from jax import lax
from jax.experimental import pallas as pl
from jax.experimental.pallas import tpu as pltpu
from jax.experimental.pallas import tpu_sc as plsc

_N = 10000
_E = 320000
_D = 128
_OUT = 64
_L = 5
_ALPHA = 0.1
_BETA = float(np.log(1.0 / 128.0 + 1.0))
_EPS = 1e-5

_NPAD = 10240
_NC, _NS = 2, 16
_NW = _NC * _NS
_EPT = _E // _NW
_K = 80
_NCH = _EPT // _K
_RPT = _NPAD // _NS
_NZB = _RPT // _K

def _agg_body(feat, src, dst3, zrows, out,
              idx_d, ia, ib, rows_a, rows_b, acc,
              sem_ia, sem_ib, sem_ga, sem_gb, sem_sa, sem_sb):
    assert _NCH % 2 == 1
    c = lax.axis_index("c")
    s = lax.axis_index("s")
    wid = s * _NC + c
    rbase = s * _RPT
    ebase = wid * _EPT
    pltpu.async_copy(dst3.at[wid], idx_d, sem_ia)
    pltpu.sync_copy(zrows, rows_a)
    for i in range(_NZB):
        pltpu.async_copy(rows_a, acc.at[pl.ds(rbase + i * _K, _K)], sem_sa)
    for i in range(_NZB):
        pltpu.make_async_copy(rows_a, acc.at[pl.ds(rbase, _K)], sem_sa).wait()
    pltpu.make_async_copy(dst3.at[wid], idx_d, sem_ia).wait()
    plsc.subcore_barrier()

    def il(j, buf, sem):
        pltpu.async_copy(src.at[pl.ds(ebase + j * _K, _K)], buf, sem)

    def il_wait(buf, sem):
        pltpu.make_async_copy(src.at[pl.ds(0, _K)], buf, sem).wait()

    def g(buf_i, buf, sem):
        pltpu.async_copy(feat.at[buf_i], buf, sem)

    def g_wait(buf, sem):
        pltpu.make_async_copy(feat.at[ia], buf, sem).wait()

    def sct(j, buf, sem):
        pltpu.async_copy(buf, acc.at[idx_d.at[j]], sem, add=True)

    def s_wait(buf, sem):
        pltpu.make_async_copy(buf, acc.at[idx_d.at[0]], sem).wait()

    il(0, ia, sem_ia)
    il_wait(ia, sem_ia)
    g(ia, rows_a, sem_ga)
    il(1, ib, sem_ib)

    def pair(t, carry):
        j0 = 2 * t
        j1 = j0 + 1
        j2 = j0 + 2
        j3 = j0 + 3
        g_wait(rows_a, sem_ga)

        @pl.when(j1 < _NCH)
        def _():
            il_wait(ib, sem_ib)

        @pl.when(t > 0)
        def _():
            s_wait(rows_b, sem_sb)

        sct(j0, rows_a, sem_sa)

        @pl.when(j1 < _NCH)
        def _():
            g(ib, rows_b, sem_gb)

        @pl.when(j2 < _NCH)
        def _():
            il(j2, ia, sem_ia)

        @pl.when(j1 < _NCH)
        def _():
            g_wait(rows_b, sem_gb)
            sct(j1, rows_b, sem_sb)

        s_wait(rows_a, sem_sa)

        @pl.when(j2 < _NCH)
        def _():
            il_wait(ia, sem_ia)
            g(ia, rows_a, sem_ga)

        @pl.when(j3 < _NCH)
        def _():
            il(j3, ib, sem_ib)

        return carry

    lax.fori_loop(0, (_NCH + 1) // 2, pair, 0)
    plsc.subcore_barrier()

    def ld(i, buf, sem):
        pltpu.async_copy(acc.at[pl.ds(rbase + i * _K, _K)], buf, sem)

    def ld_wait(buf, sem):
        pltpu.make_async_copy(acc.at[pl.ds(rbase, _K)], buf, sem).wait()

    def st(i, buf, sem):
        pltpu.async_copy(buf, out.at[c, pl.ds(rbase + i * _K, _K)], sem)

    def st_wait(buf, sem):
        pltpu.make_async_copy(buf, out.at[c, pl.ds(rbase, _K)], sem).wait()

    ld(0, rows_a, sem_ga)
    for i in range(_NZB):
        even = i % 2 == 0
        buf = rows_a if even else rows_b
        ld_wait(buf, sem_ga if even else sem_gb)
        st(i, buf, sem_sa if even else sem_sb)
        if i + 1 < _NZB:
            nbuf = rows_b if even else rows_a
            if i >= 1:
                st_wait(nbuf, sem_sb if even else sem_sa)
            ld(i + 1, nbuf, sem_gb if even else sem_ga)
    st_wait(rows_a if (_NZB - 2) % 2 == 0 else rows_b,
            sem_sa if (_NZB - 2) % 2 == 0 else sem_sb)
    st_wait(rows_a if (_NZB - 1) % 2 == 0 else rows_b,
            sem_sa if (_NZB - 1) % 2 == 0 else sem_sb)


import functools


@functools.cache
def _sc_mesh():
    return plsc.VectorSubcoreMesh(
        core_axis_name="c", subcore_axis_name="s",
        num_cores=_NC, num_subcores=_NS)


@functools.cache
def _agg_kernel():
    return pl.kernel(
        _agg_body,
        out_type=jax.ShapeDtypeStruct((_NC, _NPAD, _D), jnp.float32),
        mesh=_sc_mesh(),
        scratch_types=[
            pltpu.VMEM((_NCH, _K), jnp.int32),
            pltpu.VMEM((_K,), jnp.int32),
            pltpu.VMEM((_K,), jnp.int32),
            pltpu.VMEM((_K, _D), jnp.float32),
            pltpu.VMEM((_K, _D), jnp.float32),
            pltpu.VMEM_SHARED((_NPAD, _D), jnp.float32),
            pltpu.SemaphoreType.DMA,
            pltpu.SemaphoreType.DMA,
            pltpu.SemaphoreType.DMA,
            pltpu.SemaphoreType.DMA,
            pltpu.SemaphoreType.DMA,
            pltpu.SemaphoreType.DMA,
        ],
    )


def _agg(feat, src, dst3, zrows):
    return _agg_kernel()(feat, src, dst3, zrows)


def _deg_body(dst3, ones_h, zer_h, out, idx_d, ones_v, buf, acc,
              sem_sa, sem_sb):
    c = lax.axis_index("c")
    s = lax.axis_index("s")
    wid = s * _NC + c
    rbase = s * _RPT
    pltpu.async_copy(dst3.at[wid], idx_d, sem_sb)
    pltpu.sync_copy(zer_h, buf)
    for i in range(_NZB):
        pltpu.async_copy(buf, acc.at[pl.ds(rbase + i * _K, _K)], sem_sa)
    pltpu.sync_copy(ones_h, ones_v)
    for i in range(_NZB):
        pltpu.make_async_copy(buf, acc.at[pl.ds(rbase, _K)], sem_sa).wait()
    pltpu.make_async_copy(dst3.at[wid], idx_d, sem_sb).wait()
    plsc.subcore_barrier()

    def pair(t, carry):
        j0 = 2 * t
        j1 = j0 + 1

        @pl.when(t > 0)
        def _():
            pltpu.make_async_copy(ones_v, acc.at[idx_d.at[0]], sem_sa).wait()

        pltpu.async_copy(ones_v, acc.at[idx_d.at[j0]], sem_sa, add=True)

        @pl.when(t > 0)
        def _():
            pltpu.make_async_copy(ones_v, acc.at[idx_d.at[0]], sem_sb).wait()

        @pl.when(j1 < _NCH)
        def _():
            pltpu.async_copy(ones_v, acc.at[idx_d.at[j1]], sem_sb, add=True)

        return carry

    lax.fori_loop(0, (_NCH + 1) // 2, pair, 0)
    pltpu.make_async_copy(ones_v, acc.at[idx_d.at[0]], sem_sa).wait()
    if _NCH % 2 == 0:
        pltpu.make_async_copy(ones_v, acc.at[idx_d.at[0]], sem_sb).wait()
    plsc.subcore_barrier()
    for i in range(_NZB):
        pltpu.sync_copy(acc.at[pl.ds(rbase + i * _K, _K)], buf)
        pltpu.sync_copy(buf, out.at[c, pl.ds(rbase + i * _K, _K)])


@functools.cache
def _deg_kernel():
    return pl.kernel(
        _deg_body,
        out_type=jax.ShapeDtypeStruct((_NC, _NPAD, _D), jnp.float32),
        mesh=_sc_mesh(),
        scratch_types=[
            pltpu.VMEM((_NCH, _K), jnp.int32),
            pltpu.VMEM((_K, _D), jnp.float32),
            pltpu.VMEM((_K, _D), jnp.float32),
            pltpu.VMEM_SHARED((_NPAD, _D), jnp.float32),
            pltpu.SemaphoreType.DMA,
            pltpu.SemaphoreType.DMA,
        ],
    )


def _deg(dst3, ones_h, zer_h):
    return _deg_kernel()(dst3, ones_h, zer_h)


_R = 1000
_NB = _N // _R

_tc_params = pltpu.CompilerParams(dimension_semantics=("arbitrary",))


def _c0_body(x_ref, d0_ref, d1_ref, norm_ref, feat_ref, pool_ref):
    i = pl.program_id(0)
    deg = d0_ref[:, 0:1] + d1_ref[:, 0:1]
    nrm = lax.rsqrt(jnp.maximum(deg, 1.0))
    norm_ref[...] = nrm
    xv = x_ref[...]
    feat_ref[...] = xv * nrm

    @pl.when(i == 0)
    def _():
        pool_ref[...] = jnp.zeros_like(pool_ref)

    pool_ref[...] += jnp.sum(xv, axis=0, keepdims=True)


def _run_c0(x, deg0, deg1):
    return pl.pallas_call(
        _c0_body,
        grid=(_NB,),
        in_specs=[
            pl.BlockSpec((_R, _D), lambda i: (i, 0)),
            pl.BlockSpec((_R, _D), lambda i: (i, 0)),
            pl.BlockSpec((_R, _D), lambda i: (i, 0)),
        ],
        out_specs=[
            pl.BlockSpec((_R, 1), lambda i: (i, 0)),
            pl.BlockSpec((_R, _D), lambda i: (i, 0)),
            pl.BlockSpec((1, _D), lambda i: (0, 0)),
        ],
        out_shape=[
            jax.ShapeDtypeStruct((_NPAD, 1), jnp.float32),
            jax.ShapeDtypeStruct((_NPAD, _D), jnp.float32),
            jax.ShapeDtypeStruct((1, _D), jnp.float32),
        ],
        compiler_params=_tc_params,
    )(x, deg0, deg1)


def _c1_body(h_ref, a0_ref, a1_ref, nrm_ref, w_ref, b_ref,
             rst_ref, ssum_ref, ssq_ref):
    i = pl.program_id(0)
    ag = (a0_ref[...] + a1_ref[...]) * nrm_ref[...]
    sup = (1.0 - _ALPHA) * ag + _ALPHA * h_ref[...]
    rst = ((1.0 - _BETA) * sup
           + _BETA * jnp.dot(sup, w_ref[...],
                             preferred_element_type=jnp.float32)
           + b_ref[...])
    rst_ref[...] = rst

    @pl.when(i == 0)
    def _():
        ssum_ref[...] = jnp.zeros_like(ssum_ref)
        ssq_ref[...] = jnp.zeros_like(ssq_ref)

    ssum_ref[...] += jnp.sum(rst, axis=0, keepdims=True)
    ssq_ref[...] += jnp.sum(rst * rst, axis=0, keepdims=True)


def _run_c1(h, a0, a1, norm, w, b2):
    return pl.pallas_call(
        _c1_body,
        grid=(_NB,),
        in_specs=[
            pl.BlockSpec((_R, _D), lambda i: (i, 0)),
            pl.BlockSpec((_R, _D), lambda i: (i, 0)),
            pl.BlockSpec((_R, _D), lambda i: (i, 0)),
            pl.BlockSpec((_R, 1), lambda i: (i, 0)),
            pl.BlockSpec((_D, _D), lambda i: (0, 0)),
            pl.BlockSpec((1, _D), lambda i: (0, 0)),
        ],
        out_specs=[
            pl.BlockSpec((_R, _D), lambda i: (i, 0)),
            pl.BlockSpec((1, _D), lambda i: (0, 0)),
            pl.BlockSpec((1, _D), lambda i: (0, 0)),
        ],
        out_shape=[
            jax.ShapeDtypeStruct((_N, _D), jnp.float32),
            jax.ShapeDtypeStruct((1, _D), jnp.float32),
            jax.ShapeDtypeStruct((1, _D), jnp.float32),
        ],
        compiler_params=_tc_params,
    )(h, a0, a1, norm, w, b2)


def _c2_body(rst_ref, ssum_ref, ssq_ref, g_ref, be_ref, nrm_ref,
             h_ref, feat_ref, pool_ref):
    i = pl.program_id(0)
    mean = ssum_ref[...] * (1.0 / _N)
    var = ssq_ref[...] * (1.0 / _N) - mean * mean
    inv = lax.rsqrt(var + _EPS)
    hn = (rst_ref[...] - mean) * inv * g_ref[...] + be_ref[...]
    h = jnp.maximum(hn, 0.0)
    h_ref[...] = h
    feat_ref[...] = h * nrm_ref[...]

    @pl.when(i == 0)
    def _():
        pool_ref[...] = jnp.zeros_like(pool_ref)

    pool_ref[...] += jnp.sum(h, axis=0, keepdims=True)


def _run_c2(rst, ssum, ssq, g2, be2, norm):
    return pl.pallas_call(
        _c2_body,
        grid=(_NB,),
        in_specs=[
            pl.BlockSpec((_R, _D), lambda i: (i, 0)),
            pl.BlockSpec((1, _D), lambda i: (0, 0)),
            pl.BlockSpec((1, _D), lambda i: (0, 0)),
            pl.BlockSpec((1, _D), lambda i: (0, 0)),
            pl.BlockSpec((1, _D), lambda i: (0, 0)),
            pl.BlockSpec((_R, 1), lambda i: (i, 0)),
        ],
        out_specs=[
            pl.BlockSpec((_R, _D), lambda i: (i, 0)),
            pl.BlockSpec((_R, _D), lambda i: (i, 0)),
            pl.BlockSpec((1, _D), lambda i: (0, 0)),
        ],
        out_shape=[
            jax.ShapeDtypeStruct((_N, _D), jnp.float32),
            jax.ShapeDtypeStruct((_NPAD, _D), jnp.float32),
            jax.ShapeDtypeStruct((1, _D), jnp.float32),
        ],
        compiler_params=_tc_params,
    )(rst, ssum, ssq, g2, be2, norm)


def _d_body(pf_ref, p5_ref, lw_ref, lb_ref, out1_ref, out2_ref):
    s = (jnp.dot(pf_ref[...], lw_ref[...],
                 preferred_element_type=jnp.float32)
         + jnp.sum(lb_ref[...], axis=0, keepdims=True))
    m = jnp.max(s, axis=-1, keepdims=True)
    e = jnp.exp(s - m)
    lse = jnp.log(jnp.sum(e, axis=-1, keepdims=True))
    out1_ref[...] = s - m - lse
    out2_ref[...] = jnp.mean(p5_ref[...], axis=0, keepdims=True)


def _run_d(pf, p5, lwt, lb):
    return pl.pallas_call(
        _d_body,
        out_shape=[
            jax.ShapeDtypeStruct((1, _OUT), jnp.float32),
            jax.ShapeDtypeStruct((1, _D), jnp.float32),
        ],
    )(pf, p5, lwt, lb)


def kernel(x, edge_index, W, b, bn_gamma, bn_beta, lin_W, lin_b):
    src = edge_index[0]
    dst3 = edge_index[1].reshape(_NW, _NCH, _K)
    zrows = jnp.zeros((_K, _D), jnp.float32)
    orows = jnp.ones((_K, _D), jnp.float32)

    degp = _deg(dst3, orows, zrows)
    norm, feat, pool0 = _run_c0(x, degp[0, :_N], degp[1, :_N])

    h = x
    pooled = [pool0]
    for l in range(_L):
        aggp = _agg(feat, src, dst3, zrows)
        rst, ssum, ssq = _run_c1(h, aggp[0, :_N], aggp[1, :_N],
                                 norm[:_N], W[l], b[l][None])
        h, feat, pool = _run_c2(rst, ssum, ssq, bn_gamma[l][None],
                                bn_beta[l][None], norm[:_N])
        pooled.append(pool)

    pf = jnp.concatenate(pooled, axis=1)
    p5 = jnp.concatenate(pooled[1:], axis=0)
    lwt = jnp.transpose(lin_W, (0, 2, 1)).reshape((_L + 1) * _D, _OUT)
    out1, out2 = _run_d(pf, p5, lwt, lin_b)
    return out1, out2


# speedup vs baseline: 7.9916x; 1.0052x over previous
import numpy as np
import jax
import jax.numpy as jnp
from jax import lax
from jax.experimental import pallas as pl
from jax.experimental.pallas import tpu as pltpu
from jax.experimental.pallas import tpu_sc as plsc

_N = 10000
_E = 320000
_D = 128
_OUT = 64
_L = 5
_ALPHA = 0.1
_BETA = float(np.log(1.0 / 128.0 + 1.0))
_EPS = 1e-5

_NPAD = 10240
_NC, _NS = 2, 16
_NW = _NC * _NS
_EPT = _E // _NW
_K = 80
_NCH = _EPT // _K
_RPT = _NPAD // _NS
_NZB = _RPT // _K

def _agg_body(feat, src, dst3, zrows, out,
              idx_d, ia, ib, rows_a, rows_b, acc,
              sem_ia, sem_ib, sem_ga, sem_gb, sem_sa, sem_sb):
    assert _NCH % 2 == 1
    c = lax.axis_index("c")
    s = lax.axis_index("s")
    wid = s * _NC + c
    rbase = s * _RPT
    ebase = wid * _EPT
    pltpu.async_copy(dst3.at[wid], idx_d, sem_ia)
    pltpu.sync_copy(zrows, rows_a)
    for i in range(_NZB):
        pltpu.async_copy(rows_a, acc.at[pl.ds(rbase + i * _K, _K)], sem_sa)
    for i in range(_NZB):
        pltpu.make_async_copy(rows_a, acc.at[pl.ds(rbase, _K)], sem_sa).wait()
    pltpu.make_async_copy(dst3.at[wid], idx_d, sem_ia).wait()
    plsc.subcore_barrier()

    def il(j, buf, sem):
        pltpu.async_copy(src.at[pl.ds(ebase + j * _K, _K)], buf, sem)

    def il_wait(buf, sem):
        pltpu.make_async_copy(src.at[pl.ds(0, _K)], buf, sem).wait()

    def g(buf_i, buf, sem):
        pltpu.async_copy(feat.at[buf_i], buf, sem)

    def g_wait(buf, sem):
        pltpu.make_async_copy(feat.at[ia], buf, sem).wait()

    def sct(j, buf, sem):
        pltpu.async_copy(buf, acc.at[idx_d.at[j]], sem, add=True)

    def s_wait(buf, sem):
        pltpu.make_async_copy(buf, acc.at[idx_d.at[0]], sem).wait()

    il(0, ia, sem_ia)
    il_wait(ia, sem_ia)
    g(ia, rows_a, sem_ga)
    il(1, ib, sem_ib)

    def pair(t, carry):
        j0 = 2 * t
        j1 = j0 + 1
        j2 = j0 + 2
        j3 = j0 + 3
        g_wait(rows_a, sem_ga)

        @pl.when(j1 < _NCH)
        def _():
            il_wait(ib, sem_ib)

        @pl.when(t > 0)
        def _():
            s_wait(rows_b, sem_sb)

        sct(j0, rows_a, sem_sa)

        @pl.when(j1 < _NCH)
        def _():
            g(ib, rows_b, sem_gb)

        @pl.when(j2 < _NCH)
        def _():
            il(j2, ia, sem_ia)

        @pl.when(j1 < _NCH)
        def _():
            g_wait(rows_b, sem_gb)
            sct(j1, rows_b, sem_sb)

        s_wait(rows_a, sem_sa)

        @pl.when(j2 < _NCH)
        def _():
            il_wait(ia, sem_ia)
            g(ia, rows_a, sem_ga)

        @pl.when(j3 < _NCH)
        def _():
            il(j3, ib, sem_ib)

        return carry

    lax.fori_loop(0, (_NCH + 1) // 2, pair, 0)
    plsc.subcore_barrier()

    def ld(i, buf, sem):
        pltpu.async_copy(acc.at[pl.ds(rbase + i * _K, _K)], buf, sem)

    def ld_wait(buf, sem):
        pltpu.make_async_copy(acc.at[pl.ds(rbase, _K)], buf, sem).wait()

    def st(i, buf, sem):
        pltpu.async_copy(buf, out.at[c, pl.ds(rbase + i * _K, _K)], sem)

    def st_wait(buf, sem):
        pltpu.make_async_copy(buf, out.at[c, pl.ds(rbase, _K)], sem).wait()

    ld(0, rows_a, sem_ga)
    for i in range(_NZB):
        even = i % 2 == 0
        buf = rows_a if even else rows_b
        ld_wait(buf, sem_ga if even else sem_gb)
        st(i, buf, sem_sa if even else sem_sb)
        if i + 1 < _NZB:
            nbuf = rows_b if even else rows_a
            if i >= 1:
                st_wait(nbuf, sem_sb if even else sem_sa)
            ld(i + 1, nbuf, sem_gb if even else sem_ga)
    st_wait(rows_a if (_NZB - 2) % 2 == 0 else rows_b,
            sem_sa if (_NZB - 2) % 2 == 0 else sem_sb)
    st_wait(rows_a if (_NZB - 1) % 2 == 0 else rows_b,
            sem_sa if (_NZB - 1) % 2 == 0 else sem_sb)


import functools


@functools.cache
def _sc_mesh():
    return plsc.VectorSubcoreMesh(
        core_axis_name="c", subcore_axis_name="s",
        num_cores=_NC, num_subcores=_NS)


@functools.cache
def _agg_kernel():
    return pl.kernel(
        _agg_body,
        out_type=jax.ShapeDtypeStruct((_NC, _NPAD, _D), jnp.float32),
        mesh=_sc_mesh(),
        scratch_types=[
            pltpu.VMEM((_NCH, _K), jnp.int32),
            pltpu.VMEM((_K,), jnp.int32),
            pltpu.VMEM((_K,), jnp.int32),
            pltpu.VMEM((_K, _D), jnp.float32),
            pltpu.VMEM((_K, _D), jnp.float32),
            pltpu.VMEM_SHARED((_NPAD, _D), jnp.float32),
            pltpu.SemaphoreType.DMA,
            pltpu.SemaphoreType.DMA,
            pltpu.SemaphoreType.DMA,
            pltpu.SemaphoreType.DMA,
            pltpu.SemaphoreType.DMA,
            pltpu.SemaphoreType.DMA,
        ],
    )


def _agg(feat, src, dst3, zrows):
    return _agg_kernel()(feat, src, dst3, zrows)


def _deg_body(dst3, ones_h, zer_h, out, idx_d, ones_v, buf, acc,
              sem_sa, sem_sb):
    c = lax.axis_index("c")
    s = lax.axis_index("s")
    wid = s * _NC + c
    rbase = s * _RPT
    pltpu.async_copy(dst3.at[wid], idx_d, sem_sb)
    pltpu.sync_copy(zer_h, buf)
    for i in range(_NZB):
        pltpu.async_copy(buf, acc.at[pl.ds(rbase + i * _K, _K)], sem_sa)
    pltpu.sync_copy(ones_h, ones_v)
    for i in range(_NZB):
        pltpu.make_async_copy(buf, acc.at[pl.ds(rbase, _K)], sem_sa).wait()
    pltpu.make_async_copy(dst3.at[wid], idx_d, sem_sb).wait()
    plsc.subcore_barrier()

    def pair(t, carry):
        j0 = 2 * t
        j1 = j0 + 1

        @pl.when(t > 0)
        def _():
            pltpu.make_async_copy(ones_v, acc.at[idx_d.at[0]], sem_sa).wait()

        pltpu.async_copy(ones_v, acc.at[idx_d.at[j0]], sem_sa, add=True)

        @pl.when(t > 0)
        def _():
            pltpu.make_async_copy(ones_v, acc.at[idx_d.at[0]], sem_sb).wait()

        @pl.when(j1 < _NCH)
        def _():
            pltpu.async_copy(ones_v, acc.at[idx_d.at[j1]], sem_sb, add=True)

        return carry

    lax.fori_loop(0, (_NCH + 1) // 2, pair, 0)
    pltpu.make_async_copy(ones_v, acc.at[idx_d.at[0]], sem_sa).wait()
    if _NCH % 2 == 0:
        pltpu.make_async_copy(ones_v, acc.at[idx_d.at[0]], sem_sb).wait()
    plsc.subcore_barrier()
    for i in range(_NZB):
        pltpu.sync_copy(acc.at[pl.ds(rbase + i * _K, _K)], buf)
        pltpu.sync_copy(buf, out.at[c, pl.ds(rbase + i * _K, _K)])


@functools.cache
def _deg_kernel():
    return pl.kernel(
        _deg_body,
        out_type=jax.ShapeDtypeStruct((_NC, _NPAD, _D), jnp.float32),
        mesh=_sc_mesh(),
        scratch_types=[
            pltpu.VMEM((_NCH, _K), jnp.int32),
            pltpu.VMEM((_K, _D), jnp.float32),
            pltpu.VMEM((_K, _D), jnp.float32),
            pltpu.VMEM_SHARED((_NPAD, _D), jnp.float32),
            pltpu.SemaphoreType.DMA,
            pltpu.SemaphoreType.DMA,
        ],
    )


def _deg(dst3, ones_h, zer_h):
    return _deg_kernel()(dst3, ones_h, zer_h)


_R = 1000
_NB = _N // _R

_tc_params = pltpu.CompilerParams(dimension_semantics=("arbitrary",))


def _c0_body(x_ref, d0_ref, d1_ref, norm_ref, feat_ref, pool_ref):
    i = pl.program_id(0)
    deg = d0_ref[:, 0:1] + d1_ref[:, 0:1]
    nrm = lax.rsqrt(jnp.maximum(deg, 1.0))
    norm_ref[...] = nrm
    xv = x_ref[...]
    feat_ref[...] = xv * nrm

    @pl.when(i == 0)
    def _():
        pool_ref[...] = jnp.zeros_like(pool_ref)

    pool_ref[...] += jnp.sum(xv, axis=0, keepdims=True)


def _run_c0(x, deg0, deg1):
    return pl.pallas_call(
        _c0_body,
        grid=(_NB,),
        in_specs=[
            pl.BlockSpec((_R, _D), lambda i: (i, 0)),
            pl.BlockSpec((_R, _D), lambda i: (i, 0)),
            pl.BlockSpec((_R, _D), lambda i: (i, 0)),
        ],
        out_specs=[
            pl.BlockSpec((_R, 1), lambda i: (i, 0)),
            pl.BlockSpec((_R, _D), lambda i: (i, 0)),
            pl.BlockSpec((1, _D), lambda i: (0, 0)),
        ],
        out_shape=[
            jax.ShapeDtypeStruct((_NPAD, 1), jnp.float32),
            jax.ShapeDtypeStruct((_NPAD, _D), jnp.float32),
            jax.ShapeDtypeStruct((1, _D), jnp.float32),
        ],
        compiler_params=_tc_params,
    )(x, deg0, deg1)


def _c12_body(h_ref, a0_ref, a1_ref, nrm_ref, w_ref, b_ref, g_ref, be_ref,
              h_out, feat_ref, pool_ref, rst_scr, stats_scr):
    p = pl.program_id(0)
    i = pl.program_id(1)

    @pl.when(p == 0)
    def _():
        @pl.when(i == 0)
        def _():
            stats_scr[...] = jnp.zeros_like(stats_scr)

        ag = (a0_ref[...] + a1_ref[...]) * nrm_ref[...]
        sup = (1.0 - _ALPHA) * ag + _ALPHA * h_ref[...]
        rst = ((1.0 - _BETA) * sup
               + _BETA * jnp.dot(sup, w_ref[...],
                                 preferred_element_type=jnp.float32)
               + b_ref[...])
        rst_scr[pl.ds(i * _R, _R), :] = rst
        stats_scr[0:1, :] += jnp.sum(rst, axis=0, keepdims=True)
        stats_scr[1:2, :] += jnp.sum(rst * rst, axis=0, keepdims=True)

    @pl.when(p == 1)
    def _():
        mean = stats_scr[0:1, :] * (1.0 / _N)
        var = stats_scr[1:2, :] * (1.0 / _N) - mean * mean
        inv = lax.rsqrt(var + _EPS)
        rst = rst_scr[pl.ds(i * _R, _R), :]
        hn = (rst - mean) * inv * g_ref[...] + be_ref[...]
        h = jnp.maximum(hn, 0.0)
        h_out[...] = h
        feat_ref[...] = h * nrm_ref[...]

        @pl.when(i == 0)
        def _():
            pool_ref[...] = jnp.zeros_like(pool_ref)

        pool_ref[...] += jnp.sum(h, axis=0, keepdims=True)


def _run_c12(h, a0, a1, norm, w, b2, g2, be2):
    return pl.pallas_call(
        _c12_body,
        grid=(2, _NB),
        in_specs=[
            pl.BlockSpec((_R, _D), lambda p, i: (i, 0)),
            pl.BlockSpec((_R, _D), lambda p, i: (i, 0)),
            pl.BlockSpec((_R, _D), lambda p, i: (i, 0)),
            pl.BlockSpec((_R, 1), lambda p, i: (i, 0)),
            pl.BlockSpec((_D, _D), lambda p, i: (0, 0)),
            pl.BlockSpec((1, _D), lambda p, i: (0, 0)),
            pl.BlockSpec((1, _D), lambda p, i: (0, 0)),
            pl.BlockSpec((1, _D), lambda p, i: (0, 0)),
        ],
        out_specs=[
            pl.BlockSpec((_R, _D), lambda p, i: (p * i, 0)),
            pl.BlockSpec((_R, _D), lambda p, i: (p * i, 0)),
            pl.BlockSpec((1, _D), lambda p, i: (0, 0)),
        ],
        out_shape=[
            jax.ShapeDtypeStruct((_N, _D), jnp.float32),
            jax.ShapeDtypeStruct((_NPAD, _D), jnp.float32),
            jax.ShapeDtypeStruct((1, _D), jnp.float32),
        ],
        scratch_shapes=[
            pltpu.VMEM((_N, _D), jnp.float32),
            pltpu.VMEM((8, _D), jnp.float32),
        ],
        compiler_params=pltpu.CompilerParams(
            dimension_semantics=("arbitrary", "arbitrary")),
    )(h, a0, a1, norm, w, b2, g2, be2)


def _d_body(pf_ref, p5_ref, lw_ref, lb_ref, out1_ref, out2_ref):
    s = (jnp.dot(pf_ref[...], lw_ref[...],
                 preferred_element_type=jnp.float32)
         + jnp.sum(lb_ref[...], axis=0, keepdims=True))
    m = jnp.max(s, axis=-1, keepdims=True)
    e = jnp.exp(s - m)
    lse = jnp.log(jnp.sum(e, axis=-1, keepdims=True))
    out1_ref[...] = s - m - lse
    out2_ref[...] = jnp.mean(p5_ref[...], axis=0, keepdims=True)


def _run_d(pf, p5, lwt, lb):
    return pl.pallas_call(
        _d_body,
        out_shape=[
            jax.ShapeDtypeStruct((1, _OUT), jnp.float32),
            jax.ShapeDtypeStruct((1, _D), jnp.float32),
        ],
    )(pf, p5, lwt, lb)


def kernel(x, edge_index, W, b, bn_gamma, bn_beta, lin_W, lin_b):
    src = edge_index[0]
    dst3 = edge_index[1].reshape(_NW, _NCH, _K)
    zrows = jnp.zeros((_K, _D), jnp.float32)
    orows = jnp.ones((_K, _D), jnp.float32)

    degp = _deg(dst3, orows, zrows)
    norm, feat, pool0 = _run_c0(x, degp[0, :_N], degp[1, :_N])

    h = x
    pooled = [pool0]
    for l in range(_L):
        aggp = _agg(feat, src, dst3, zrows)
        h, feat, pool = _run_c12(h, aggp[0, :_N], aggp[1, :_N], norm[:_N],
                                 W[l], b[l][None], bn_gamma[l][None],
                                 bn_beta[l][None])
        pooled.append(pool)

    pf = jnp.concatenate(pooled, axis=1)
    p5 = jnp.concatenate(pooled[1:], axis=0)
    lwt = jnp.transpose(lin_W, (0, 2, 1)).reshape((_L + 1) * _D, _OUT)
    out1, out2 = _run_d(pf, p5, lwt, lin_b)
    return out1, out2


# speedup vs baseline: 8.0020x; 1.0013x over previous
import numpy as np
import jax
import jax.numpy as jnp
from jax import lax
from jax.experimental import pallas as pl
from jax.experimental.pallas import tpu as pltpu
from jax.experimental.pallas import tpu_sc as plsc

_N = 10000
_E = 320000
_D = 128
_OUT = 64
_L = 5
_ALPHA = 0.1
_BETA = float(np.log(1.0 / 128.0 + 1.0))
_EPS = 1e-5

_NPAD = 10240
_NC, _NS = 2, 16
_NW = _NC * _NS
_EPT = _E // _NW
_K = 80
_NCH = _EPT // _K
_RPT = _NPAD // _NS
_NZB = _RPT // _K

def _agg_body(feat, src, dst3, zrows, out,
              idx_d, ia, ib, rows_a, rows_b, acc,
              sem_ia, sem_ib, sem_ga, sem_gb, sem_sa, sem_sb):
    assert _NCH % 2 == 1
    c = lax.axis_index("c")
    s = lax.axis_index("s")
    wid = s * _NC + c
    rbase = s * _RPT
    ebase = wid * _EPT
    pltpu.async_copy(dst3.at[wid], idx_d, sem_ia)
    pltpu.sync_copy(zrows, rows_a)
    for i in range(_NZB):
        pltpu.async_copy(rows_a, acc.at[pl.ds(rbase + i * _K, _K)], sem_sa)
    for i in range(_NZB):
        pltpu.make_async_copy(rows_a, acc.at[pl.ds(rbase, _K)], sem_sa).wait()
    pltpu.make_async_copy(dst3.at[wid], idx_d, sem_ia).wait()
    plsc.subcore_barrier()

    def il(j, buf, sem):
        pltpu.async_copy(src.at[pl.ds(ebase + j * _K, _K)], buf, sem)

    def il_wait(buf, sem):
        pltpu.make_async_copy(src.at[pl.ds(0, _K)], buf, sem).wait()

    def g(buf_i, buf, sem):
        pltpu.async_copy(feat.at[buf_i], buf, sem)

    def g_wait(buf, sem):
        pltpu.make_async_copy(feat.at[ia], buf, sem).wait()

    def sct(j, buf, sem):
        pltpu.async_copy(buf, acc.at[idx_d.at[j]], sem, add=True)

    def s_wait(buf, sem):
        pltpu.make_async_copy(buf, acc.at[idx_d.at[0]], sem).wait()

    il(0, ia, sem_ia)
    il_wait(ia, sem_ia)
    g(ia, rows_a, sem_ga)
    il(1, ib, sem_ib)

    def pair(t, carry):
        j0 = 2 * t
        j1 = j0 + 1
        j2 = j0 + 2
        j3 = j0 + 3
        g_wait(rows_a, sem_ga)
        sct(j0, rows_a, sem_sa)

        @pl.when(j1 < _NCH)
        def _():
            il_wait(ib, sem_ib)

        @pl.when(t > 0)
        def _():
            s_wait(rows_b, sem_sb)

        @pl.when(j1 < _NCH)
        def _():
            g(ib, rows_b, sem_gb)

        @pl.when(j2 < _NCH)
        def _():
            il(j2, ia, sem_ia)

        @pl.when(j1 < _NCH)
        def _():
            g_wait(rows_b, sem_gb)
            sct(j1, rows_b, sem_sb)

        s_wait(rows_a, sem_sa)

        @pl.when(j2 < _NCH)
        def _():
            il_wait(ia, sem_ia)
            g(ia, rows_a, sem_ga)

        @pl.when(j3 < _NCH)
        def _():
            il(j3, ib, sem_ib)

        return carry

    lax.fori_loop(0, (_NCH + 1) // 2, pair, 0)
    plsc.subcore_barrier()

    def ld(i, buf, sem):
        pltpu.async_copy(acc.at[pl.ds(rbase + i * _K, _K)], buf, sem)

    def ld_wait(buf, sem):
        pltpu.make_async_copy(acc.at[pl.ds(rbase, _K)], buf, sem).wait()

    def st(i, buf, sem):
        pltpu.async_copy(buf, out.at[c, pl.ds(rbase + i * _K, _K)], sem)

    def st_wait(buf, sem):
        pltpu.make_async_copy(buf, out.at[c, pl.ds(rbase, _K)], sem).wait()

    ld(0, rows_a, sem_ga)
    for i in range(_NZB):
        even = i % 2 == 0
        buf = rows_a if even else rows_b
        ld_wait(buf, sem_ga if even else sem_gb)
        st(i, buf, sem_sa if even else sem_sb)
        if i + 1 < _NZB:
            nbuf = rows_b if even else rows_a
            if i >= 1:
                st_wait(nbuf, sem_sb if even else sem_sa)
            ld(i + 1, nbuf, sem_gb if even else sem_ga)
    st_wait(rows_a if (_NZB - 2) % 2 == 0 else rows_b,
            sem_sa if (_NZB - 2) % 2 == 0 else sem_sb)
    st_wait(rows_a if (_NZB - 1) % 2 == 0 else rows_b,
            sem_sa if (_NZB - 1) % 2 == 0 else sem_sb)


import functools


@functools.cache
def _sc_mesh():
    return plsc.VectorSubcoreMesh(
        core_axis_name="c", subcore_axis_name="s",
        num_cores=_NC, num_subcores=_NS)


@functools.cache
def _agg_kernel():
    return pl.kernel(
        _agg_body,
        out_type=jax.ShapeDtypeStruct((_NC, _NPAD, _D), jnp.float32),
        mesh=_sc_mesh(),
        scratch_types=[
            pltpu.VMEM((_NCH, _K), jnp.int32),
            pltpu.VMEM((_K,), jnp.int32),
            pltpu.VMEM((_K,), jnp.int32),
            pltpu.VMEM((_K, _D), jnp.float32),
            pltpu.VMEM((_K, _D), jnp.float32),
            pltpu.VMEM_SHARED((_NPAD, _D), jnp.float32),
            pltpu.SemaphoreType.DMA,
            pltpu.SemaphoreType.DMA,
            pltpu.SemaphoreType.DMA,
            pltpu.SemaphoreType.DMA,
            pltpu.SemaphoreType.DMA,
            pltpu.SemaphoreType.DMA,
        ],
    )


def _agg(feat, src, dst3, zrows):
    return _agg_kernel()(feat, src, dst3, zrows)


def _deg_body(dst3, ones_h, zer_h, out, idx_d, ones_v, buf, acc,
              sem_sa, sem_sb):
    c = lax.axis_index("c")
    s = lax.axis_index("s")
    wid = s * _NC + c
    rbase = s * _RPT
    pltpu.async_copy(dst3.at[wid], idx_d, sem_sb)
    pltpu.sync_copy(zer_h, buf)
    for i in range(_NZB):
        pltpu.async_copy(buf, acc.at[pl.ds(rbase + i * _K, _K)], sem_sa)
    pltpu.sync_copy(ones_h, ones_v)
    for i in range(_NZB):
        pltpu.make_async_copy(buf, acc.at[pl.ds(rbase, _K)], sem_sa).wait()
    pltpu.make_async_copy(dst3.at[wid], idx_d, sem_sb).wait()
    plsc.subcore_barrier()

    def pair(t, carry):
        j0 = 2 * t
        j1 = j0 + 1

        @pl.when(t > 0)
        def _():
            pltpu.make_async_copy(ones_v, acc.at[idx_d.at[0]], sem_sa).wait()

        pltpu.async_copy(ones_v, acc.at[idx_d.at[j0]], sem_sa, add=True)

        @pl.when(t > 0)
        def _():
            pltpu.make_async_copy(ones_v, acc.at[idx_d.at[0]], sem_sb).wait()

        @pl.when(j1 < _NCH)
        def _():
            pltpu.async_copy(ones_v, acc.at[idx_d.at[j1]], sem_sb, add=True)

        return carry

    lax.fori_loop(0, (_NCH + 1) // 2, pair, 0)
    pltpu.make_async_copy(ones_v, acc.at[idx_d.at[0]], sem_sa).wait()
    if _NCH % 2 == 0:
        pltpu.make_async_copy(ones_v, acc.at[idx_d.at[0]], sem_sb).wait()
    plsc.subcore_barrier()
    for i in range(_NZB):
        pltpu.sync_copy(acc.at[pl.ds(rbase + i * _K, _K)], buf)
        pltpu.sync_copy(buf, out.at[c, pl.ds(rbase + i * _K, _K)])


@functools.cache
def _deg_kernel():
    return pl.kernel(
        _deg_body,
        out_type=jax.ShapeDtypeStruct((_NC, _NPAD, _D), jnp.float32),
        mesh=_sc_mesh(),
        scratch_types=[
            pltpu.VMEM((_NCH, _K), jnp.int32),
            pltpu.VMEM((_K, _D), jnp.float32),
            pltpu.VMEM((_K, _D), jnp.float32),
            pltpu.VMEM_SHARED((_NPAD, _D), jnp.float32),
            pltpu.SemaphoreType.DMA,
            pltpu.SemaphoreType.DMA,
        ],
    )


def _deg(dst3, ones_h, zer_h):
    return _deg_kernel()(dst3, ones_h, zer_h)


_R = 1000
_NB = _N // _R

_tc_params = pltpu.CompilerParams(dimension_semantics=("arbitrary",))


def _c0_body(x_ref, d0_ref, d1_ref, norm_ref, feat_ref, pool_ref):
    i = pl.program_id(0)
    deg = d0_ref[:, 0:1] + d1_ref[:, 0:1]
    nrm = lax.rsqrt(jnp.maximum(deg, 1.0))
    norm_ref[...] = nrm
    xv = x_ref[...]
    feat_ref[...] = xv * nrm

    @pl.when(i == 0)
    def _():
        pool_ref[...] = jnp.zeros_like(pool_ref)

    pool_ref[...] += jnp.sum(xv, axis=0, keepdims=True)


def _run_c0(x, deg0, deg1):
    return pl.pallas_call(
        _c0_body,
        grid=(_NB,),
        in_specs=[
            pl.BlockSpec((_R, _D), lambda i: (i, 0)),
            pl.BlockSpec((_R, _D), lambda i: (i, 0)),
            pl.BlockSpec((_R, _D), lambda i: (i, 0)),
        ],
        out_specs=[
            pl.BlockSpec((_R, 1), lambda i: (i, 0)),
            pl.BlockSpec((_R, _D), lambda i: (i, 0)),
            pl.BlockSpec((1, _D), lambda i: (0, 0)),
        ],
        out_shape=[
            jax.ShapeDtypeStruct((_NPAD, 1), jnp.float32),
            jax.ShapeDtypeStruct((_NPAD, _D), jnp.float32),
            jax.ShapeDtypeStruct((1, _D), jnp.float32),
        ],
        compiler_params=_tc_params,
    )(x, deg0, deg1)


def _c12_body(h_ref, a0_ref, a1_ref, nrm_ref, w_ref, b_ref, g_ref, be_ref,
              h_out, feat_ref, pool_ref, rst_scr, stats_scr):
    p = pl.program_id(0)
    i = pl.program_id(1)

    @pl.when(p == 0)
    def _():
        @pl.when(i == 0)
        def _():
            stats_scr[...] = jnp.zeros_like(stats_scr)

        ag = (a0_ref[...] + a1_ref[...]) * nrm_ref[...]
        sup = (1.0 - _ALPHA) * ag + _ALPHA * h_ref[...]
        rst = ((1.0 - _BETA) * sup
               + _BETA * jnp.dot(sup, w_ref[...],
                                 preferred_element_type=jnp.float32)
               + b_ref[...])
        rst_scr[pl.ds(i * _R, _R), :] = rst
        stats_scr[0:1, :] += jnp.sum(rst, axis=0, keepdims=True)
        stats_scr[1:2, :] += jnp.sum(rst * rst, axis=0, keepdims=True)

    @pl.when(p == 1)
    def _():
        mean = stats_scr[0:1, :] * (1.0 / _N)
        var = stats_scr[1:2, :] * (1.0 / _N) - mean * mean
        inv = lax.rsqrt(var + _EPS)
        rst = rst_scr[pl.ds(i * _R, _R), :]
        hn = (rst - mean) * inv * g_ref[...] + be_ref[...]
        h = jnp.maximum(hn, 0.0)
        h_out[...] = h
        feat_ref[...] = h * nrm_ref[...]

        @pl.when(i == 0)
        def _():
            pool_ref[...] = jnp.zeros_like(pool_ref)

        pool_ref[...] += jnp.sum(h, axis=0, keepdims=True)


def _run_c12(h, a0, a1, norm, w, b2, g2, be2):
    return pl.pallas_call(
        _c12_body,
        grid=(2, _NB),
        in_specs=[
            pl.BlockSpec((_R, _D), lambda p, i: (i, 0)),
            pl.BlockSpec((_R, _D), lambda p, i: (i, 0)),
            pl.BlockSpec((_R, _D), lambda p, i: (i, 0)),
            pl.BlockSpec((_R, 1), lambda p, i: (i, 0)),
            pl.BlockSpec((_D, _D), lambda p, i: (0, 0)),
            pl.BlockSpec((1, _D), lambda p, i: (0, 0)),
            pl.BlockSpec((1, _D), lambda p, i: (0, 0)),
            pl.BlockSpec((1, _D), lambda p, i: (0, 0)),
        ],
        out_specs=[
            pl.BlockSpec((_R, _D), lambda p, i: (p * i, 0)),
            pl.BlockSpec((_R, _D), lambda p, i: (p * i, 0)),
            pl.BlockSpec((1, _D), lambda p, i: (0, 0)),
        ],
        out_shape=[
            jax.ShapeDtypeStruct((_N, _D), jnp.float32),
            jax.ShapeDtypeStruct((_NPAD, _D), jnp.float32),
            jax.ShapeDtypeStruct((1, _D), jnp.float32),
        ],
        scratch_shapes=[
            pltpu.VMEM((_N, _D), jnp.float32),
            pltpu.VMEM((8, _D), jnp.float32),
        ],
        compiler_params=pltpu.CompilerParams(
            dimension_semantics=("arbitrary", "arbitrary")),
    )(h, a0, a1, norm, w, b2, g2, be2)


def _d_body(pf_ref, p5_ref, lw_ref, lb_ref, out1_ref, out2_ref):
    s = (jnp.dot(pf_ref[...], lw_ref[...],
                 preferred_element_type=jnp.float32)
         + jnp.sum(lb_ref[...], axis=0, keepdims=True))
    m = jnp.max(s, axis=-1, keepdims=True)
    e = jnp.exp(s - m)
    lse = jnp.log(jnp.sum(e, axis=-1, keepdims=True))
    out1_ref[...] = s - m - lse
    out2_ref[...] = jnp.mean(p5_ref[...], axis=0, keepdims=True)


def _run_d(pf, p5, lwt, lb):
    return pl.pallas_call(
        _d_body,
        out_shape=[
            jax.ShapeDtypeStruct((1, _OUT), jnp.float32),
            jax.ShapeDtypeStruct((1, _D), jnp.float32),
        ],
    )(pf, p5, lwt, lb)


def kernel(x, edge_index, W, b, bn_gamma, bn_beta, lin_W, lin_b):
    src = edge_index[0]
    dst3 = edge_index[1].reshape(_NW, _NCH, _K)
    zrows = jnp.zeros((_K, _D), jnp.float32)
    orows = jnp.ones((_K, _D), jnp.float32)

    degp = _deg(dst3, orows, zrows)
    norm, feat, pool0 = _run_c0(x, degp[0, :_N], degp[1, :_N])

    h = x
    pooled = [pool0]
    for l in range(_L):
        aggp = _agg(feat, src, dst3, zrows)
        h, feat, pool = _run_c12(h, aggp[0, :_N], aggp[1, :_N], norm[:_N],
                                 W[l], b[l][None], bn_gamma[l][None],
                                 bn_beta[l][None])
        pooled.append(pool)

    pf = jnp.concatenate(pooled, axis=1)
    p5 = jnp.concatenate(pooled[1:], axis=0)
    lwt = jnp.transpose(lin_W, (0, 2, 1)).reshape((_L + 1) * _D, _OUT)
    out1, out2 = _run_d(pf, p5, lwt, lin_b)
    return out1, out2


# speedup vs baseline: 8.3461x; 1.0430x over previous
import numpy as np
import jax
import jax.numpy as jnp
from jax import lax
from jax.experimental import pallas as pl
from jax.experimental.pallas import tpu as pltpu
from jax.experimental.pallas import tpu_sc as plsc

_N = 10000
_E = 320000
_D = 128
_OUT = 64
_L = 5
_ALPHA = 0.1
_BETA = float(np.log(1.0 / 128.0 + 1.0))
_EPS = 1e-5

_NPAD = 10240
_NC, _NS = 2, 16
_NW = _NC * _NS
_EPT = _E // _NW
_K = 80
_NCH = _EPT // _K
_RPT = _NPAD // _NS
_NZB = _RPT // _K

def _agg_body(feat, src, dst3, zrows, out,
              idx_d, ia, ib, rows_a, rows_b, acc,
              sem_ia, sem_ib, sem_ga, sem_gb, sem_sa, sem_sb):
    assert _NCH % 2 == 1
    c = lax.axis_index("c")
    s = lax.axis_index("s")
    wid = s * _NC + c
    rbase = s * _RPT
    ebase = wid * _EPT
    pltpu.async_copy(dst3.at[wid], idx_d, sem_ia)
    pltpu.sync_copy(zrows, rows_a)
    for i in range(_NZB):
        pltpu.async_copy(rows_a, acc.at[pl.ds(rbase + i * _K, _K)], sem_sa)
    for i in range(_NZB):
        pltpu.make_async_copy(rows_a, acc.at[pl.ds(rbase, _K)], sem_sa).wait()
    pltpu.make_async_copy(dst3.at[wid], idx_d, sem_ia).wait()
    plsc.subcore_barrier()

    def il(j, buf, sem):
        pltpu.async_copy(src.at[pl.ds(ebase + j * _K, _K)], buf, sem)

    def il_wait(buf, sem):
        pltpu.make_async_copy(src.at[pl.ds(0, _K)], buf, sem).wait()

    def g(buf_i, buf, sem):
        pltpu.async_copy(feat.at[buf_i], buf, sem)

    def g_wait(buf, sem):
        pltpu.make_async_copy(feat.at[ia], buf, sem).wait()

    def sct(j, buf, sem):
        pltpu.async_copy(buf, acc.at[idx_d.at[j]], sem, add=True)

    def s_wait(buf, sem):
        pltpu.make_async_copy(buf, acc.at[idx_d.at[0]], sem).wait()

    il(0, ia, sem_ia)
    il_wait(ia, sem_ia)
    g(ia, rows_a, sem_ga)
    il(1, ib, sem_ib)

    def pair(t, carry):
        j0 = 2 * t
        j1 = j0 + 1
        j2 = j0 + 2
        j3 = j0 + 3
        g_wait(rows_a, sem_ga)
        sct(j0, rows_a, sem_sa)

        @pl.when(j1 < _NCH)
        def _():
            il_wait(ib, sem_ib)

        @pl.when(t > 0)
        def _():
            s_wait(rows_b, sem_sb)

        @pl.when(j1 < _NCH)
        def _():
            g(ib, rows_b, sem_gb)

        @pl.when(j2 < _NCH)
        def _():
            il(j2, ia, sem_ia)

        @pl.when(j1 < _NCH)
        def _():
            g_wait(rows_b, sem_gb)
            sct(j1, rows_b, sem_sb)

        s_wait(rows_a, sem_sa)

        @pl.when(j2 < _NCH)
        def _():
            il_wait(ia, sem_ia)
            g(ia, rows_a, sem_ga)

        @pl.when(j3 < _NCH)
        def _():
            il(j3, ib, sem_ib)

        return carry

    lax.fori_loop(0, (_NCH + 1) // 2, pair, 0)
    plsc.subcore_barrier()

    def ld(i, buf, sem):
        pltpu.async_copy(acc.at[pl.ds(rbase + i * _K, _K)], buf, sem)

    def ld_wait(buf, sem):
        pltpu.make_async_copy(acc.at[pl.ds(rbase, _K)], buf, sem).wait()

    def st(i, buf, sem):
        pltpu.async_copy(buf, out.at[c, pl.ds(rbase + i * _K, _K)], sem)

    def st_wait(buf, sem):
        pltpu.make_async_copy(buf, out.at[c, pl.ds(rbase, _K)], sem).wait()

    ld(0, rows_a, sem_ga)
    for i in range(_NZB):
        even = i % 2 == 0
        buf = rows_a if even else rows_b
        ld_wait(buf, sem_ga if even else sem_gb)
        st(i, buf, sem_sa if even else sem_sb)
        if i + 1 < _NZB:
            nbuf = rows_b if even else rows_a
            if i >= 1:
                st_wait(nbuf, sem_sb if even else sem_sa)
            ld(i + 1, nbuf, sem_gb if even else sem_ga)
    st_wait(rows_a if (_NZB - 2) % 2 == 0 else rows_b,
            sem_sa if (_NZB - 2) % 2 == 0 else sem_sb)
    st_wait(rows_a if (_NZB - 1) % 2 == 0 else rows_b,
            sem_sa if (_NZB - 1) % 2 == 0 else sem_sb)


import functools


@functools.cache
def _sc_mesh():
    return plsc.VectorSubcoreMesh(
        core_axis_name="c", subcore_axis_name="s",
        num_cores=_NC, num_subcores=_NS)


@functools.cache
def _agg_kernel():
    return pl.kernel(
        _agg_body,
        out_type=jax.ShapeDtypeStruct((_NC, _NPAD, _D), jnp.float32),
        mesh=_sc_mesh(),
        scratch_types=[
            pltpu.VMEM((_NCH, _K), jnp.int32),
            pltpu.VMEM((_K,), jnp.int32),
            pltpu.VMEM((_K,), jnp.int32),
            pltpu.VMEM((_K, _D), jnp.float32),
            pltpu.VMEM((_K, _D), jnp.float32),
            pltpu.VMEM_SHARED((_NPAD, _D), jnp.float32),
            pltpu.SemaphoreType.DMA,
            pltpu.SemaphoreType.DMA,
            pltpu.SemaphoreType.DMA,
            pltpu.SemaphoreType.DMA,
            pltpu.SemaphoreType.DMA,
            pltpu.SemaphoreType.DMA,
        ],
    )


def _agg(feat, src, dst3, zrows):
    return _agg_kernel()(feat, src, dst3, zrows)


def _deg_body(dst3, ones_h, zer_h, out, idx_d, ones_v, buf, acc,
              sem_sa, sem_sb):
    c = lax.axis_index("c")
    s = lax.axis_index("s")
    wid = s * _NC + c
    rbase = s * _RPT
    pltpu.async_copy(dst3.at[wid], idx_d, sem_sb)
    pltpu.sync_copy(zer_h, buf)
    for i in range(_NZB):
        pltpu.async_copy(buf, acc.at[pl.ds(rbase + i * _K, _K)], sem_sa)
    pltpu.sync_copy(ones_h, ones_v)
    for i in range(_NZB):
        pltpu.make_async_copy(buf, acc.at[pl.ds(rbase, _K)], sem_sa).wait()
    pltpu.make_async_copy(dst3.at[wid], idx_d, sem_sb).wait()
    plsc.subcore_barrier()

    def pair(t, carry):
        j0 = 2 * t
        j1 = j0 + 1

        @pl.when(t > 0)
        def _():
            pltpu.make_async_copy(ones_v, acc.at[idx_d.at[0]], sem_sa).wait()

        pltpu.async_copy(ones_v, acc.at[idx_d.at[j0]], sem_sa, add=True)

        @pl.when(t > 0)
        def _():
            pltpu.make_async_copy(ones_v, acc.at[idx_d.at[0]], sem_sb).wait()

        @pl.when(j1 < _NCH)
        def _():
            pltpu.async_copy(ones_v, acc.at[idx_d.at[j1]], sem_sb, add=True)

        return carry

    lax.fori_loop(0, (_NCH + 1) // 2, pair, 0)
    pltpu.make_async_copy(ones_v, acc.at[idx_d.at[0]], sem_sa).wait()
    if _NCH % 2 == 0:
        pltpu.make_async_copy(ones_v, acc.at[idx_d.at[0]], sem_sb).wait()
    plsc.subcore_barrier()
    for i in range(_NZB):
        pltpu.sync_copy(acc.at[pl.ds(rbase + i * _K, _K)], buf)
        pltpu.sync_copy(buf, out.at[c, pl.ds(rbase + i * _K, _K)])


@functools.cache
def _deg_kernel():
    return pl.kernel(
        _deg_body,
        out_type=jax.ShapeDtypeStruct((_NC, _NPAD, _D), jnp.float32),
        mesh=_sc_mesh(),
        scratch_types=[
            pltpu.VMEM((_NCH, _K), jnp.int32),
            pltpu.VMEM((_K, _D), jnp.float32),
            pltpu.VMEM((_K, _D), jnp.float32),
            pltpu.VMEM_SHARED((_NPAD, _D), jnp.float32),
            pltpu.SemaphoreType.DMA,
            pltpu.SemaphoreType.DMA,
        ],
    )


def _deg(dst3, ones_h, zer_h):
    return _deg_kernel()(dst3, ones_h, zer_h)


_R = 1000
_NB = _N // _R

_tc_params = pltpu.CompilerParams(dimension_semantics=("arbitrary",))


def _c0_body(x_ref, d0_ref, d1_ref, norm_ref, feat_ref, pool_ref):
    i = pl.program_id(0)
    deg = d0_ref[0, :, 0:1] + d1_ref[0, :, 0:1]
    nrm = lax.rsqrt(jnp.maximum(deg, 1.0))
    norm_ref[...] = nrm
    xv = x_ref[...]
    feat_ref[...] = xv * nrm

    @pl.when(i == 0)
    def _():
        pool_ref[...] = jnp.zeros_like(pool_ref)

    pool_ref[...] += jnp.sum(xv, axis=0, keepdims=True)


def _run_c0(x, degp):
    return pl.pallas_call(
        _c0_body,
        grid=(_NB,),
        in_specs=[
            pl.BlockSpec((_R, _D), lambda i: (i, 0)),
            pl.BlockSpec((1, _R, _D), lambda i: (0, i, 0)),
            pl.BlockSpec((1, _R, _D), lambda i: (1, i, 0)),
        ],
        out_specs=[
            pl.BlockSpec((_R, 1), lambda i: (i, 0)),
            pl.BlockSpec((_R, _D), lambda i: (i, 0)),
            pl.BlockSpec((1, _D), lambda i: (0, 0)),
        ],
        out_shape=[
            jax.ShapeDtypeStruct((_NPAD, 1), jnp.float32),
            jax.ShapeDtypeStruct((_NPAD, _D), jnp.float32),
            jax.ShapeDtypeStruct((1, _D), jnp.float32),
        ],
        compiler_params=_tc_params,
    )(x, degp, degp)


def _c12_body(h_ref, a0_ref, a1_ref, nrm_ref, w_ref, b_ref, g_ref, be_ref,
              h_out, feat_ref, pool_ref, rst_scr, stats_scr):
    p = pl.program_id(0)
    i = pl.program_id(1)

    @pl.when(p == 0)
    def _():
        @pl.when(i == 0)
        def _():
            stats_scr[...] = jnp.zeros_like(stats_scr)

        ag = (a0_ref[0] + a1_ref[0]) * nrm_ref[...]
        sup = (1.0 - _ALPHA) * ag + _ALPHA * h_ref[...]
        rst = ((1.0 - _BETA) * sup
               + _BETA * jnp.dot(sup, w_ref[...],
                                 preferred_element_type=jnp.float32)
               + b_ref[...])
        rst_scr[pl.ds(i * _R, _R), :] = rst
        stats_scr[0:1, :] += jnp.sum(rst, axis=0, keepdims=True)
        stats_scr[1:2, :] += jnp.sum(rst * rst, axis=0, keepdims=True)

    @pl.when(p == 1)
    def _():
        mean = stats_scr[0:1, :] * (1.0 / _N)
        var = stats_scr[1:2, :] * (1.0 / _N) - mean * mean
        inv = lax.rsqrt(var + _EPS)
        rst = rst_scr[pl.ds(i * _R, _R), :]
        hn = (rst - mean) * inv * g_ref[...] + be_ref[...]
        h = jnp.maximum(hn, 0.0)
        h_out[...] = h
        feat_ref[...] = h * nrm_ref[...]

        @pl.when(i == 0)
        def _():
            pool_ref[...] = jnp.zeros_like(pool_ref)

        pool_ref[...] += jnp.sum(h, axis=0, keepdims=True)


def _run_c12(h, aggp, norm, w, b2, g2, be2):
    return pl.pallas_call(
        _c12_body,
        grid=(2, _NB),
        in_specs=[
            pl.BlockSpec((_R, _D), lambda p, i: (i, 0)),
            pl.BlockSpec((1, _R, _D), lambda p, i: (0, i, 0)),
            pl.BlockSpec((1, _R, _D), lambda p, i: (1, i, 0)),
            pl.BlockSpec((_R, 1), lambda p, i: (i, 0)),
            pl.BlockSpec((_D, _D), lambda p, i: (0, 0)),
            pl.BlockSpec((1, _D), lambda p, i: (0, 0)),
            pl.BlockSpec((1, _D), lambda p, i: (0, 0)),
            pl.BlockSpec((1, _D), lambda p, i: (0, 0)),
        ],
        out_specs=[
            pl.BlockSpec((_R, _D), lambda p, i: (p * i, 0)),
            pl.BlockSpec((_R, _D), lambda p, i: (p * i, 0)),
            pl.BlockSpec((1, _D), lambda p, i: (0, 0)),
        ],
        out_shape=[
            jax.ShapeDtypeStruct((_N, _D), jnp.float32),
            jax.ShapeDtypeStruct((_NPAD, _D), jnp.float32),
            jax.ShapeDtypeStruct((1, _D), jnp.float32),
        ],
        scratch_shapes=[
            pltpu.VMEM((_N, _D), jnp.float32),
            pltpu.VMEM((8, _D), jnp.float32),
        ],
        compiler_params=pltpu.CompilerParams(
            dimension_semantics=("arbitrary", "arbitrary")),
    )(h, aggp, aggp, norm, w, b2, g2, be2)


def _d_body(pf_ref, p5_ref, lw_ref, lb_ref, out1_ref, out2_ref):
    s = (jnp.dot(pf_ref[...], lw_ref[...],
                 preferred_element_type=jnp.float32)
         + jnp.sum(lb_ref[...], axis=0, keepdims=True))
    m = jnp.max(s, axis=-1, keepdims=True)
    e = jnp.exp(s - m)
    lse = jnp.log(jnp.sum(e, axis=-1, keepdims=True))
    out1_ref[...] = s - m - lse
    out2_ref[...] = jnp.mean(p5_ref[...], axis=0, keepdims=True)


def _run_d(pf, p5, lwt, lb):
    return pl.pallas_call(
        _d_body,
        out_shape=[
            jax.ShapeDtypeStruct((1, _OUT), jnp.float32),
            jax.ShapeDtypeStruct((1, _D), jnp.float32),
        ],
    )(pf, p5, lwt, lb)


def kernel(x, edge_index, W, b, bn_gamma, bn_beta, lin_W, lin_b):
    src = edge_index[0]
    dst3 = edge_index[1].reshape(_NW, _NCH, _K)
    zrows = jnp.zeros((_K, _D), jnp.float32)
    orows = jnp.ones((_K, _D), jnp.float32)

    degp = _deg(dst3, orows, zrows)
    norm, feat, pool0 = _run_c0(x, degp)

    h = x
    pooled = [pool0]
    for l in range(_L):
        aggp = _agg(feat, src, dst3, zrows)
        h, feat, pool = _run_c12(h, aggp, norm, W[l], b[l][None],
                                 bn_gamma[l][None], bn_beta[l][None])
        pooled.append(pool)

    pf = jnp.concatenate(pooled, axis=1)
    p5 = jnp.concatenate(pooled[1:], axis=0)
    lwt = jnp.transpose(lin_W, (0, 2, 1)).reshape((_L + 1) * _D, _OUT)
    out1, out2 = _run_d(pf, p5, lwt, lin_b)
    return out1, out2
